# baseline (speedup 1.0000x reference)
import functools

import jax
import jax.numpy as jnp
from jax.experimental import pallas as pl
from jax.experimental.pallas import tpu as pltpu
from jax.experimental.pallas import tpu_sc as plsc

_T = 2048
_D = 1024
_E = 8
_K = 2
_F = 4096
_EPS = 1e-5

_BM = 128
_NT = _T * _K // _BM + _E
_P = _NT * _BM
_FB = 512
_NF = _F // _FB

_NEG = jnp.finfo(jnp.float32).min


def _dispatch_body(x_ref, mask_ref, lnw_ref, wr_ref, tril_ref,
                   pos_ref, pos2_ref, p2_ref, tg_ref, act_ref):
    x = x_ref[...]
    var = jnp.mean(x * x, axis=-1, keepdims=True)
    xn = x * jax.lax.rsqrt(var + _EPS) * lnw_ref[...]
    logits = jnp.dot(xn, wr_ref[...], preferred_element_type=jnp.float32)
    mask = mask_ref[...] != 0
    logits = jnp.where(mask, logits, _NEG)

    ii = jax.lax.broadcasted_iota(jnp.int32, logits.shape, 1)
    m1 = jnp.max(logits, axis=-1, keepdims=True)
    i1 = jnp.min(jnp.where(logits == m1, ii, _E), axis=-1, keepdims=True)
    is1 = ii == i1
    l2 = jnp.where(is1, _NEG, logits)
    m2 = jnp.max(l2, axis=-1, keepdims=True)
    i2 = jnp.min(jnp.where(l2 == m2, ii, _E), axis=-1, keepdims=True)
    is2 = ii == i2
    hard = is1 | is2

    z = jnp.exp(logits - m1)
    probs = z / jnp.sum(z, axis=-1, keepdims=True)
    probs = jnp.where(hard & mask, probs, 0.0)

    sel = hard.astype(jnp.bfloat16)
    rank = jnp.dot(tril_ref[...], sel, preferred_element_type=jnp.float32)
    rank = rank.astype(jnp.int32)
    counts = jnp.sum(hard.astype(jnp.float32), axis=0, keepdims=True)
    counts = counts.astype(jnp.int32)
    c_pad = ((counts + _BM - 1) // _BM) * _BM
    ei = jax.lax.broadcasted_iota(jnp.int32, (_E, _E), 0)
    ej = jax.lax.broadcasted_iota(jnp.int32, (_E, _E), 1)
    upper = (ei < ej).astype(jnp.bfloat16)
    off = jnp.dot(c_pad.astype(jnp.bfloat16), upper,
                  preferred_element_type=jnp.float32).astype(jnp.int32)
    ends = off + c_pad

    pos = off + rank
    pos_ref[...] = jnp.where(hard, pos, _P)

    pos1 = jnp.sum(jnp.where(is1, pos, 0), axis=-1, keepdims=True)
    pos2_ = jnp.sum(jnp.where(is2, pos, 0), axis=-1, keepdims=True)
    pos2_ref[...] = jnp.concatenate([pos1, pos2_], axis=-1)
    pb1 = jnp.sum(jnp.where(is1, probs, 0.0), axis=-1, keepdims=True)
    pb2 = jnp.sum(jnp.where(is2, probs, 0.0), axis=-1, keepdims=True)
    p2_ref[...] = jnp.concatenate([pb1, pb2], axis=-1)

    ms = jax.lax.broadcasted_iota(jnp.int32, (_NT, _E), 0) * _BM
    tg = jnp.sum((ms >= ends).astype(jnp.int32), axis=-1, keepdims=True)
    tg_ref[...] = jnp.minimum(tg, _E - 1)
    mc = jax.lax.broadcasted_iota(jnp.int32, (_NT, 1), 0) * _BM
    act_ref[...] = (mc < ends[:, _E - 1:_E]).astype(jnp.int32)


def _dispatch(x, mask, ln_w, w_router, tril):
    return pl.pallas_call(
        _dispatch_body,
        out_shape=[
            jax.ShapeDtypeStruct((_T, _E), jnp.int32),
            jax.ShapeDtypeStruct((_T, _K), jnp.int32),
            jax.ShapeDtypeStruct((_T, _K), jnp.float32),
            jax.ShapeDtypeStruct((_NT, 1), jnp.int32),
            jax.ShapeDtypeStruct((_NT, 1), jnp.int32),
        ],
    )(x, mask.astype(jnp.int32).reshape(_T, 1), ln_w.reshape(1, _D),
      w_router, tril)


def _gmm_body(tg_ref, act_ref, xg_ref, w1_ref, w2_ref, y_ref, acc_ref, xb_ref):
    f = pl.program_id(0)
    s = pl.program_id(1)

    @pl.when(act_ref[s] == 1)
    def _():
        @pl.when(f == 0)
        def _():
            xb_ref[pl.ds(s * _BM, _BM), :] = xg_ref[...].astype(jnp.bfloat16)

        xb = xb_ref[pl.ds(s * _BM, _BM), :]
        h = jnp.dot(xb, w1_ref[0], preferred_element_type=jnp.float32)
        h = jax.nn.gelu(h)
        contrib = jnp.dot(h.astype(jnp.bfloat16), w2_ref[0],
                          preferred_element_type=jnp.float32)

        @pl.when(f == 0)
        def _():
            acc_ref[pl.ds(s * _BM, _BM), :] = contrib

        @pl.when((f > 0) & (f < _NF - 1))
        def _():
            acc_ref[pl.ds(s * _BM, _BM), :] += contrib

        @pl.when(f == _NF - 1)
        def _():
            y_ref[...] = acc_ref[pl.ds(s * _BM, _BM), :] + contrib


def _gmm(xg, w1_bf, w2_bf, tg, act):
    grid_spec = pltpu.PrefetchScalarGridSpec(
        num_scalar_prefetch=2,
        grid=(_NF, _NT),
        in_specs=[
            pl.BlockSpec((_BM, _D),
                         lambda f, s, tg, act: (jnp.where(f == 0, s, 0), 0)),
            pl.BlockSpec((1, _D, _FB), lambda f, s, tg, act: (tg[s], 0, f)),
            pl.BlockSpec((1, _FB, _D), lambda f, s, tg, act: (tg[s], f, 0)),
        ],
        out_specs=pl.BlockSpec(
            (_BM, _D), lambda f, s, tg, act: (jnp.where(f == _NF - 1, s, 0), 0)),
        scratch_shapes=[
            pltpu.VMEM((_P, _D), jnp.float32),
            pltpu.VMEM((_P, _D), jnp.bfloat16),
        ],
    )
    return pl.pallas_call(
        _gmm_body,
        grid_spec=grid_spec,
        out_shape=jax.ShapeDtypeStruct((_P, _D), jnp.float32),
    )(tg, act, xg, w1_bf, w2_bf)


def kernel(x, mask, ln_w, w_router, w1, w2):
    tril = jnp.tri(_T, _T, -1, dtype=jnp.bfloat16)
    pos, pos2, p2, tg, act = _dispatch(x, mask, ln_w, w_router, tril)
    tg = tg.reshape(_NT)
    act = act.reshape(_NT)

    pos_flat = pos.reshape(_T * _E)
    tok = jnp.arange(_T * _E, dtype=jnp.int32) // _E
    tok_sorted = jnp.zeros((_P + 8,), jnp.int32).at[pos_flat].set(
        tok, mode="drop", unique_indices=False)
    idx = jnp.clip(tok_sorted[:_P], 0, _T - 1)
    xg = x[idx]

    y = _gmm(xg, w1.astype(jnp.bfloat16), w2.astype(jnp.bfloat16), tg, act)

    y0 = y[jnp.clip(pos2[:, 0], 0, _P - 1)]
    y1 = y[jnp.clip(pos2[:, 1], 0, _P - 1)]
    out = x + p2[:, 0:1] * y0 + p2[:, 1:2] * y1
    return out
